# baseline (speedup 1.0000x reference)
import functools
import math

import jax
import jax.numpy as jnp
from jax import lax
from jax.experimental import pallas as pl
from jax.experimental.pallas import tpu as pltpu
from jax.experimental.pallas import tpu_sc as plsc

_VOCAB = 100000
_N = 1024
_SMOOTH = 0.1
_CONF = 1.0 - _SMOOTH
_S = _SMOOTH / (_VOCAB - 2)
_C0 = (_VOCAB - 2) * _S * math.log(_S) + _CONF * math.log(_CONF)

_ROWS_PER_STEP = 64

_NC, _NS, _L = 2, 16, 16
_NW = _NC * _NS
_ROWS_PER_W = _N // _NW


def _sc_gather_terms(pred_flat, tgt):
    mesh = plsc.VectorSubcoreMesh(core_axis_name="c", subcore_axis_name="s")

    @functools.partial(
        pl.kernel,
        mesh=mesh,
        out_type=jax.ShapeDtypeStruct((_N,), jnp.float32),
        scratch_types=[
            pltpu.VMEM((_ROWS_PER_W,), jnp.int32),
            pltpu.VMEM((_ROWS_PER_W,), jnp.int32),
            pltpu.VMEM((_ROWS_PER_W,), jnp.float32),
            pltpu.VMEM((_ROWS_PER_W,), jnp.float32),
            pltpu.SemaphoreType.DMA,
        ],
    )
    def k(pred_hbm, t_hbm, out_hbm, t_v, idx_v, g_v, r_v, sem):
        wid = lax.axis_index("s") * _NC + lax.axis_index("c")
        base = wid * _ROWS_PER_W
        pltpu.sync_copy(t_hbm.at[pl.ds(base, _ROWS_PER_W)], t_v)
        for j in range(_ROWS_PER_W // _L):
            t16 = t_v[pl.ds(j * _L, _L)]
            row16 = (base + j * _L) + lax.iota(jnp.int32, _L)
            idx_v[pl.ds(j * _L, _L)] = row16 * _VOCAB + t16
        pltpu.async_copy(pred_hbm.at[idx_v], g_v, sem).wait()
        for j in range(_ROWS_PER_W // _L):
            t16 = t_v[pl.ds(j * _L, _L)]
            g16 = g_v[pl.ds(j * _L, _L)]
            r_v[pl.ds(j * _L, _L)] = jnp.where(
                t16 != 0, jnp.float32(_S - _CONF) * g16, jnp.float32(0.0)
            )
        pltpu.sync_copy(r_v, out_hbm.at[pl.ds(base, _ROWS_PER_W)])

    return k(pred_flat, tgt)


def _tc_reduce(pred, t2d, a2d):
    nsteps = _N // _ROWS_PER_STEP

    def body(x_ref, t_ref, a_ref, out_ref):
        i = pl.program_id(0)
        x = x_ref[...]
        maskf = (t_ref[...] != 0).astype(jnp.float32)
        rowsum = jnp.sum(x, axis=1, keepdims=True)
        x0 = x[:, 0:1]
        partial = jnp.sum(
            maskf * (jnp.float32(_C0) + jnp.float32(_S) * (x0 - rowsum))
            + a_ref[...]
        )

        @pl.when(i == 0)
        def _():
            out_ref[0, 0] = partial

        @pl.when(i > 0)
        def _():
            out_ref[0, 0] += partial

    return pl.pallas_call(
        body,
        grid=(nsteps,),
        in_specs=[
            pl.BlockSpec((_ROWS_PER_STEP, _VOCAB), lambda i: (i, 0)),
            pl.BlockSpec((_ROWS_PER_STEP, 1), lambda i: (i, 0)),
            pl.BlockSpec((_ROWS_PER_STEP, 1), lambda i: (i, 0)),
        ],
        out_specs=pl.BlockSpec(
            (1, 1), lambda i: (0, 0), memory_space=pltpu.SMEM
        ),
        out_shape=jax.ShapeDtypeStruct((1, 1), jnp.float32),
    )(pred, t2d, a2d)


def kernel(pred_scores, target_ids):
    t = target_ids.astype(jnp.int32)
    a = _sc_gather_terms(pred_scores.reshape(-1), t)
    out = _tc_reduce(pred_scores, t.reshape(_N, 1), a.reshape(_N, 1))
    return out[0, 0]


# speedup vs baseline: 2.1080x; 2.1080x over previous
import jax
import jax.numpy as jnp
import math
from jax import lax
from jax.experimental import pallas as pl
from jax.experimental.pallas import tpu as pltpu

_VOCAB = 100000
_N = 1024
_SMOOTH = 0.1
_CONF = 1.0 - _SMOOTH
_S = _SMOOTH / (_VOCAB - 2)
_C0 = (_VOCAB - 2) * _S * math.log(_S) + _CONF * math.log(_CONF)

_ROWS_PER_STEP = 32


def _tc_fused(pred, t2d):
    nsteps = _N // _ROWS_PER_STEP

    def body(x_ref, t_ref, out_ref):
        i = pl.program_id(0)
        x = x_ref[...]
        t = t_ref[...]
        maskf = (t != 0).astype(jnp.float32)
        xm = x * maskf
        col = lax.broadcasted_iota(jnp.int32, (_ROWS_PER_STEP, _VOCAB), 1)
        sum_all = jnp.sum(xm)
        sum_col0 = jnp.sum(xm[:, 0:1])
        sum_tgt = jnp.sum(jnp.where(col == t, xm, jnp.float32(0.0)))
        partial = (
            jnp.float32(_C0) * jnp.sum(maskf)
            - jnp.float32(_S) * sum_all
            + jnp.float32(_S) * sum_col0
            + jnp.float32(_S - _CONF) * sum_tgt
        )

        @pl.when(i == 0)
        def _():
            out_ref[0, 0] = partial

        @pl.when(i > 0)
        def _():
            out_ref[0, 0] += partial

    return pl.pallas_call(
        body,
        grid=(nsteps,),
        in_specs=[
            pl.BlockSpec((_ROWS_PER_STEP, _VOCAB), lambda i: (i, 0)),
            pl.BlockSpec((_ROWS_PER_STEP, 1), lambda i: (i, 0)),
        ],
        out_specs=pl.BlockSpec(
            (1, 1), lambda i: (0, 0), memory_space=pltpu.SMEM
        ),
        out_shape=jax.ShapeDtypeStruct((1, 1), jnp.float32),
    )(pred, t2d)


def kernel(pred_scores, target_ids):
    t = target_ids.astype(jnp.int32)
    out = _tc_fused(pred_scores, t.reshape(_N, 1))
    return out[0, 0]


# speedup vs baseline: 8.5587x; 4.0600x over previous
import jax
import jax.numpy as jnp
import math
from jax import lax
from jax.experimental import pallas as pl
from jax.experimental.pallas import tpu as pltpu

_VOCAB = 100000
_N = 1024
_SMOOTH = 0.1
_CONF = 1.0 - _SMOOTH
_S = _SMOOTH / (_VOCAB - 2)
_C0 = (_VOCAB - 2) * _S * math.log(_S) + _CONF * math.log(_CONF)

_CB = 2000


def _tc_fused_t(xt, t1):
    nsteps = _VOCAB // _CB

    def body(x_ref, t_ref, out_ref, acc_ref):
        k = pl.program_id(0)
        x = x_ref[...]
        t = t_ref[...]
        rowid = lax.broadcasted_iota(jnp.int32, (_CB, _N), 0) + k * _CB
        part = jnp.float32(-_S) * jnp.sum(x, axis=0, keepdims=True) + jnp.float32(
            _S - _CONF
        ) * jnp.sum(jnp.where(rowid == t, x, jnp.float32(0.0)), axis=0, keepdims=True)

        @pl.when(k == 0)
        def _():
            acc_ref[...] = part + jnp.float32(_S) * x[0:1, :]

        @pl.when(k > 0)
        def _():
            acc_ref[...] += part

        @pl.when(k == nsteps - 1)
        def _():
            maskf = (t != 0).astype(jnp.float32)
            out_ref[0, 0] = jnp.sum(maskf * acc_ref[...]) + jnp.float32(
                _C0
            ) * jnp.sum(maskf)

    return pl.pallas_call(
        body,
        grid=(nsteps,),
        in_specs=[
            pl.BlockSpec((_CB, _N), lambda k: (k, 0)),
            pl.BlockSpec((1, _N), lambda k: (0, 0)),
        ],
        out_specs=pl.BlockSpec(
            (1, 1), lambda k: (0, 0), memory_space=pltpu.SMEM
        ),
        out_shape=jax.ShapeDtypeStruct((1, 1), jnp.float32),
        scratch_shapes=[pltpu.VMEM((1, _N), jnp.float32)],
    )(xt, t1)


def kernel(pred_scores, target_ids):
    xt = pred_scores.T
    t1 = target_ids.astype(jnp.int32).reshape(1, _N)
    out = _tc_fused_t(xt, t1)
    return out[0, 0]


# speedup vs baseline: 8.7397x; 1.0211x over previous
import jax
import jax.numpy as jnp
import math
from jax import lax
from jax.experimental import pallas as pl
from jax.experimental.pallas import tpu as pltpu

_VOCAB = 100000
_N = 1024
_SMOOTH = 0.1
_CONF = 1.0 - _SMOOTH
_S = _SMOOTH / (_VOCAB - 2)
_C0 = (_VOCAB - 2) * _S * math.log(_S) + _CONF * math.log(_CONF)

_CB = 4000


def _tc_fused_t(xt, t1):
    nsteps = _VOCAB // _CB

    def body(x_ref, t_ref, out_ref, acc_ref):
        k = pl.program_id(0)
        x = x_ref[...]
        t = t_ref[...]
        rowid = lax.broadcasted_iota(jnp.int32, (_CB, _N), 0) + k * _CB
        part = jnp.float32(-_S) * jnp.sum(x, axis=0, keepdims=True) + jnp.float32(
            _S - _CONF
        ) * jnp.sum(jnp.where(rowid == t, x, jnp.float32(0.0)), axis=0, keepdims=True)

        @pl.when(k == 0)
        def _():
            acc_ref[...] = part + jnp.float32(_S) * x[0:1, :]

        @pl.when(k > 0)
        def _():
            acc_ref[...] += part

        @pl.when(k == nsteps - 1)
        def _():
            maskf = (t != 0).astype(jnp.float32)
            out_ref[0, 0] = jnp.sum(maskf * acc_ref[...]) + jnp.float32(
                _C0
            ) * jnp.sum(maskf)

    return pl.pallas_call(
        body,
        grid=(nsteps,),
        in_specs=[
            pl.BlockSpec((_CB, _N), lambda k: (k, 0)),
            pl.BlockSpec((1, _N), lambda k: (0, 0)),
        ],
        out_specs=pl.BlockSpec(
            (1, 1), lambda k: (0, 0), memory_space=pltpu.SMEM
        ),
        out_shape=jax.ShapeDtypeStruct((1, 1), jnp.float32),
        scratch_shapes=[pltpu.VMEM((1, _N), jnp.float32)],
    )(xt, t1)


def kernel(pred_scores, target_ids):
    xt = pred_scores.T
    t1 = target_ids.astype(jnp.int32).reshape(1, _N)
    out = _tc_fused_t(xt, t1)
    return out[0, 0]
